# baseline (speedup 1.0000x reference)
import functools
import math

import jax
import jax.numpy as jnp
from jax import lax
from jax.experimental import pallas as pl
from jax.experimental.pallas import tpu as pltpu
from jax.experimental.pallas import tpu_sc as plsc

B, P, L, H = 4, 256, 200, 2048
NIMG = 3
IMG_SEQ = NIMG * P
SEQ = IMG_SEQ + L
SCALE = math.sqrt(H)

NC, NS = 2, 16
NW = NC * NS
CHUNKS_PER_B = NW // B
TOK_PER_W = L // CHUNKS_PER_B
TOK_PAD = 32
LANES = 16


def _sc_gather_body(tok_hbm, table_hbm, out_hbm, idx_v, rows_v, sem):
    c = lax.axis_index("c")
    s = lax.axis_index("s")
    wid = s * NC + c
    b = wid // CHUNKS_PER_B
    ch = wid - b * CHUNKS_PER_B
    pltpu.sync_copy(tok_hbm.at[wid], idx_v)
    pltpu.async_copy(table_hbm.at[idx_v], rows_v, sem).wait()

    def scale_row(r, carry):
        for j in range(H // LANES):
            sl = pl.ds(j * LANES, LANES)
            rows_v[r, sl] = rows_v[r, sl] * SCALE
        return carry

    lax.fori_loop(0, TOK_PER_W, scale_row, 0)

    dst = b * SEQ + IMG_SEQ + ch * TOK_PER_W
    pltpu.sync_copy(rows_v.at[pl.ds(0, TOK_PER_W)],
                    out_hbm.at[pl.ds(dst, TOK_PER_W)])


@jax.jit
def _sc_gather(tok2, table):
    return pl.kernel(
        _sc_gather_body,
        out_type=jax.ShapeDtypeStruct((B * SEQ, H), jnp.float32),
        mesh=plsc.VectorSubcoreMesh(core_axis_name="c", subcore_axis_name="s"),
        scratch_types=[
            pltpu.VMEM((TOK_PAD,), jnp.int32),
            pltpu.VMEM((TOK_PAD, H), jnp.float32),
            pltpu.SemaphoreType.DMA,
        ],
        compiler_params=pltpu.CompilerParams(use_tc_tiling_on_sc=False),
    )(tok2, table)


def _tc_fill_body(img0, img1, img2, buf_in, out, sem):
    del buf_in
    copies = []
    for i, img in enumerate((img0, img1, img2)):
        for b in range(B):
            copies.append(pltpu.make_async_copy(
                img.at[b], out.at[pl.ds(b * SEQ + i * P, P)], sem))
    for cp in copies:
        cp.start()
    for cp in copies:
        cp.wait()


@jax.jit
def _tc_fill(img0, img1, img2, buf):
    return pl.pallas_call(
        _tc_fill_body,
        out_shape=jax.ShapeDtypeStruct((B * SEQ, H), jnp.float32),
        in_specs=[pl.BlockSpec(memory_space=pl.ANY)] * 4,
        out_specs=pl.BlockSpec(memory_space=pl.ANY),
        scratch_shapes=[pltpu.SemaphoreType.DMA],
        input_output_aliases={3: 0},
    )(img0, img1, img2, buf)


def kernel(img_emb_0, img_emb_1, img_emb_2, img_mask_0, img_mask_1,
           img_mask_2, lang_tokens, lang_masks, embed_table):
    tok2 = lang_tokens.reshape(B, CHUNKS_PER_B, TOK_PER_W)
    tok2 = jnp.pad(tok2, ((0, 0), (0, 0), (0, TOK_PAD - TOK_PER_W)))
    tok2 = tok2.reshape(NW, TOK_PAD)

    buf = _sc_gather(tok2, embed_table)
    buf = _tc_fill(img_emb_0, img_emb_1, img_emb_2, buf)
    prefix_embs = buf.reshape(B, SEQ, H)

    pad_img = jnp.concatenate([
        jnp.broadcast_to(m[:, None], (B, P))
        for m in (img_mask_0, img_mask_1, img_mask_2)
    ], axis=1)
    prefix_pad_masks = jnp.concatenate([pad_img, lang_masks], axis=1)
    prefix_att_masks = jnp.zeros_like(prefix_pad_masks)
    return prefix_embs, prefix_pad_masks, prefix_att_masks, SEQ
